# baseline (speedup 1.0000x reference)
import functools

import jax
import jax.numpy as jnp
from jax import lax
from jax.experimental import pallas as pl
from jax.experimental.pallas import tpu as pltpu
from jax.experimental.pallas import tpu_sc as plsc

_NC = 2
_NS = 16
_NW = _NC * _NS
_L = 16

_NACC = 10
_CHUNK = 20000


def _argmax_sc(probs_flat, num_rows, vocab):
    rows_per_w = num_rows // _NW
    nchunk = vocab // _CHUNK
    inner_iters = _CHUNK // (_L * _NACC)
    mesh = plsc.VectorSubcoreMesh(
        core_axis_name="c", subcore_axis_name="s",
        num_cores=_NC, num_subcores=_NS)

    @functools.partial(
        pl.kernel,
        out_type=jax.ShapeDtypeStruct((num_rows,), jnp.int32),
        mesh=mesh,
        scratch_types=[
            pltpu.VMEM((_CHUNK,), jnp.float32),
            pltpu.VMEM((rows_per_w,), jnp.int32),
        ],
    )
    def body(probs_hbm, out_hbm, buf, outv):
        c = lax.axis_index("c")
        s = lax.axis_index("s")
        wid = c * _NS + s
        lane = lax.broadcasted_iota(jnp.int32, (_L,), 0)
        neg = jnp.full((_L,), -1.0, jnp.float32)
        zero = jnp.zeros((_L,), jnp.int32)
        big = jnp.full((_L,), 2**30, jnp.int32)

        def row_body(r, res):
            row = wid * rows_per_w + r
            row_off = row * vocab

            def chunk_body(k, carry):
                avs = carry[:_NACC]
                ais = carry[_NACC:]
                pltpu.sync_copy(
                    probs_hbm.at[pl.ds(row_off + k * _CHUNK, _CHUNK)], buf)
                ixs = tuple(lane + (k * _CHUNK + _L * j) for j in range(_NACC))

                def vec_body(i, c2):
                    vs = list(c2[:_NACC])
                    vi = list(c2[_NACC:2 * _NACC])
                    vx = list(c2[2 * _NACC:])
                    base = i * (_L * _NACC)
                    for j in range(_NACC):
                        x = buf[pl.ds(base + _L * j, _L)]
                        m = x > vs[j]
                        vs[j] = jnp.where(m, x, vs[j])
                        vi[j] = jnp.where(m, vx[j], vi[j])
                        vx[j] = vx[j] + (_L * _NACC)
                    return tuple(vs) + tuple(vi) + tuple(vx)

                c2 = lax.fori_loop(0, inner_iters, vec_body,
                                   tuple(avs) + tuple(ais) + ixs)
                return c2[:2 * _NACC]

            init = tuple([neg] * _NACC) + tuple([zero] * _NACC)
            fin = lax.fori_loop(0, nchunk, chunk_body, init)
            avs, ais = fin[:_NACC], fin[_NACC:]
            lm = avs[0]
            for j in range(1, _NACC):
                lm = jnp.maximum(lm, avs[j])
            row_max = lm
            for sh in (1, 2, 4, 8):
                row_max = jnp.maximum(
                    row_max,
                    row_max.at[lane ^ sh].get(mode='promise_in_bounds'))
            cm = big
            for j in range(_NACC):
                cm = jnp.minimum(cm, jnp.where(avs[j] == row_max, ais[j], big))
            idx = cm
            for sh in (1, 2, 4, 8):
                idx = jnp.minimum(
                    idx, idx.at[lane ^ sh].get(mode='promise_in_bounds'))
            return jnp.where(lane == r, idx, res)

        res = lax.fori_loop(0, rows_per_w, row_body, zero)
        outv[...] = res
        pltpu.sync_copy(outv, out_hbm.at[pl.ds(wid * rows_per_w, rows_per_w)])

    return body(probs_flat)


def _epilogue_tc(tam2d, draft2d, cu2d, bonus, spec):
    batch = tam2d.shape[0]
    sentinel = 2 * spec

    def body(tam_ref, draft_ref, cu_ref, bonus_ref, out_ref):
        tam = tam_ref[...]
        draft = draft_ref[...]
        cu = cu_ref[...]
        bns = bonus_ref[...]
        cu_prev = jnp.concatenate(
            [jnp.zeros((1, 1), jnp.int32), cu[:-1, :]], axis=0)
        dpr = cu - cu_prev
        pos = lax.broadcasted_iota(jnp.int32, (batch, spec), 1)
        mm_pos = jnp.where(draft != tam, pos, sentinel)
        fm = jnp.min(mm_pos, axis=1, keepdims=True)
        fm = jnp.where(fm == sentinel, dpr, fm)
        copy_len = jnp.minimum(fm + 1, dpr)
        main = jnp.where(pos < copy_len, tam, -1)
        bcol = jnp.where(fm >= dpr, bns, -1)
        out_ref[...] = jnp.concatenate([main, bcol], axis=1)

    return pl.pallas_call(
        body,
        out_shape=jax.ShapeDtypeStruct((batch, spec + 1), jnp.int32),
    )(tam2d, draft2d, cu2d, bonus)


def kernel(draft_token_ids, cu_num_draft_tokens, target_probs,
           bonus_token_ids, max_spec_len):
    num_tokens, vocab = target_probs.shape
    batch = cu_num_draft_tokens.shape[0]
    spec = num_tokens // batch
    tam = _argmax_sc(target_probs.reshape(-1), num_tokens, vocab)
    return _epilogue_tc(
        tam.reshape(batch, spec),
        draft_token_ids.reshape(batch, spec),
        cu_num_draft_tokens.reshape(batch, 1).astype(jnp.int32),
        bonus_token_ids.astype(jnp.int32),
        spec)


# speedup vs baseline: 1.1968x; 1.1968x over previous
import functools

import jax
import jax.numpy as jnp
from jax import lax
from jax.experimental import pallas as pl
from jax.experimental.pallas import tpu as pltpu
from jax.experimental.pallas import tpu_sc as plsc

_NC = 2
_NS = 16
_NW = _NC * _NS
_L = 16

_NACC = 10
_CHUNK = 20000


def _argmax_sc(probs_flat, num_rows, vocab):
    rows_per_w = num_rows // _NW
    nchunk = vocab // _CHUNK
    inner_iters = _CHUNK // (_L * _NACC)
    nch_total = rows_per_w * nchunk
    stride = _L * _NACC
    mesh = plsc.VectorSubcoreMesh(
        core_axis_name="c", subcore_axis_name="s",
        num_cores=_NC, num_subcores=_NS)

    @functools.partial(
        pl.kernel,
        out_type=jax.ShapeDtypeStruct((num_rows,), jnp.int32),
        mesh=mesh,
        scratch_types=[
            pltpu.VMEM((_CHUNK,), jnp.float32),
            pltpu.VMEM((_CHUNK,), jnp.float32),
            pltpu.VMEM((rows_per_w,), jnp.int32),
            pltpu.SemaphoreType.DMA,
            pltpu.SemaphoreType.DMA,
        ],
    )
    def body(probs_hbm, out_hbm, buf0, buf1, outv, sem0, sem1):
        c = lax.axis_index("c")
        s = lax.axis_index("s")
        wid = c * _NS + s
        lane = lax.broadcasted_iota(jnp.int32, (_L,), 0)
        neg = jnp.full((_L,), -1.0, jnp.float32)
        zero = jnp.zeros((_L,), jnp.int32)
        big = jnp.full((_L,), 2**30, jnp.int32)
        shuf = tuple(lane ^ sh for sh in (1, 2, 4, 8))
        row_base = wid * rows_per_w

        def chunk_off(g):
            gc = jnp.minimum(g, nch_total - 1)
            return (row_base + gc // nchunk) * vocab + (gc % nchunk) * _CHUNK

        pltpu.async_copy(
            probs_hbm.at[pl.ds(chunk_off(0), _CHUNK)], buf0, sem0)

        def process(g, buf, sem, nbuf, nsem, carry):
            pltpu.async_copy(
                probs_hbm.at[pl.ds(chunk_off(g + 1), _CHUNK)], nbuf, nsem)
            pltpu.make_async_copy(
                probs_hbm.at[pl.ds(chunk_off(g), _CHUNK)], buf, sem).wait()
            res = carry[0]
            k = g % nchunk
            is_first = lane == lane + k
            avs = [jnp.where(is_first, neg, v) for v in carry[1:1 + _NACC]]
            ais = [jnp.where(is_first, zero, v) for v in carry[1 + _NACC:]]
            tbase = k * inner_iters

            def vec_body(i, c2):
                vs = list(c2[:_NACC])
                vi = list(c2[_NACC:])
                tsplat = jnp.broadcast_to(tbase + i, (_L,))
                base = i * stride
                for j in range(_NACC):
                    x = buf[pl.ds(base + _L * j, _L)]
                    m = x > vs[j]
                    vs[j] = jnp.where(m, x, vs[j])
                    vi[j] = jnp.where(m, tsplat, vi[j])
                return tuple(vs) + tuple(vi)

            c2 = lax.fori_loop(0, inner_iters, vec_body,
                               tuple(avs) + tuple(ais))
            avs, ais = c2[:_NACC], c2[_NACC:]
            lm = avs[0]
            for j in range(1, _NACC):
                lm = jnp.maximum(lm, avs[j])
            row_max = lm
            for sv in shuf:
                row_max = jnp.maximum(
                    row_max, row_max.at[sv].get(mode='promise_in_bounds'))
            cm = big
            for j in range(_NACC):
                idx_j = ais[j] * stride + (_L * j) + lane
                cm = jnp.minimum(
                    cm, jnp.where(avs[j] == row_max, idx_j, big))
            for sv in shuf:
                cm = jnp.minimum(
                    cm, cm.at[sv].get(mode='promise_in_bounds'))
            res = jnp.where(lane == lane + (k - (nchunk - 1)),
                            jnp.where(lane == g // nchunk, cm, res), res)
            return (res,) + tuple(avs) + tuple(ais)

        def outer(gg, carry):
            carry = process(2 * gg, buf0, sem0, buf1, sem1, carry)
            carry = process(2 * gg + 1, buf1, sem1, buf0, sem0, carry)
            return carry

        init = (zero,) + tuple([neg] * _NACC) + tuple([zero] * _NACC)
        fin = lax.fori_loop(0, nch_total // 2, outer, init)
        pltpu.make_async_copy(
            probs_hbm.at[pl.ds(chunk_off(nch_total), _CHUNK)], buf0,
            sem0).wait()
        outv[...] = fin[0]
        pltpu.sync_copy(outv, out_hbm.at[pl.ds(wid * rows_per_w, rows_per_w)])

    return body(probs_flat)


def _epilogue_tc(tam2d, draft2d, cu2d, bonus, spec):
    batch = tam2d.shape[0]
    sentinel = 2 * spec

    def body(tam_ref, draft_ref, cu_ref, bonus_ref, out_ref):
        tam = tam_ref[...]
        draft = draft_ref[...]
        cu = cu_ref[...]
        bns = bonus_ref[...]
        cu_prev = jnp.concatenate(
            [jnp.zeros((1, 1), jnp.int32), cu[:-1, :]], axis=0)
        dpr = cu - cu_prev
        pos = lax.broadcasted_iota(jnp.int32, (batch, spec), 1)
        mm_pos = jnp.where(draft != tam, pos, sentinel)
        fm = jnp.min(mm_pos, axis=1, keepdims=True)
        fm = jnp.where(fm == sentinel, dpr, fm)
        copy_len = jnp.minimum(fm + 1, dpr)
        main = jnp.where(pos < copy_len, tam, -1)
        bcol = jnp.where(fm >= dpr, bns, -1)
        out_ref[...] = jnp.concatenate([main, bcol], axis=1)

    return pl.pallas_call(
        body,
        out_shape=jax.ShapeDtypeStruct((batch, spec + 1), jnp.int32),
    )(tam2d, draft2d, cu2d, bonus)


def kernel(draft_token_ids, cu_num_draft_tokens, target_probs,
           bonus_token_ids, max_spec_len):
    num_tokens, vocab = target_probs.shape
    batch = cu_num_draft_tokens.shape[0]
    spec = num_tokens // batch
    tam = _argmax_sc(target_probs.reshape(-1), num_tokens, vocab)
    return _epilogue_tc(
        tam.reshape(batch, spec),
        draft_token_ids.reshape(batch, spec),
        cu_num_draft_tokens.reshape(batch, 1).astype(jnp.int32),
        bonus_token_ids.astype(jnp.int32),
        spec)


# speedup vs baseline: 2.2303x; 1.8635x over previous
import functools

import jax
import jax.numpy as jnp
from jax import lax
from jax.experimental import pallas as pl
from jax.experimental.pallas import tpu as pltpu
from jax.experimental.pallas import tpu_sc as plsc

_NC = 2
_NS = 16
_NW = _NC * _NS
_L = 16

_TR = 8
_CCOLS = 3072


def _argmax_sc(probs, num_rows, ncols):
    rowtiles_per_w = num_rows // (_NW * _TR)
    nchunk = ncols // _CCOLS
    inner_iters = _CCOLS // _L
    rows_per_w = rowtiles_per_w * _TR
    mesh = plsc.VectorSubcoreMesh(
        core_axis_name="c", subcore_axis_name="s",
        num_cores=_NC, num_subcores=_NS)

    @functools.partial(
        pl.kernel,
        out_type=(jax.ShapeDtypeStruct((num_rows,), jnp.int32),
                  jax.ShapeDtypeStruct((num_rows,), jnp.float32)),
        mesh=mesh,
        scratch_types=[
            pltpu.VMEM((_TR, _CCOLS), jnp.float32),
            pltpu.VMEM((_TR, _CCOLS), jnp.float32),
            pltpu.VMEM((rows_per_w,), jnp.int32),
            pltpu.VMEM((rows_per_w,), jnp.float32),
            pltpu.SemaphoreType.DMA,
            pltpu.SemaphoreType.DMA,
        ],
        compiler_params=pltpu.CompilerParams(use_tc_tiling_on_sc=True),
    )
    def body(probs_hbm, out_idx, out_max, buf0, buf1, outv, outm, sem0, sem1):
        c = lax.axis_index("c")
        s = lax.axis_index("s")
        wid = c * _NS + s
        lane = lax.broadcasted_iota(jnp.int32, (_L,), 0)
        neg = jnp.full((_L,), -1.0, jnp.float32)
        zero = jnp.zeros((_L,), jnp.int32)
        big = jnp.full((_L,), 2**30, jnp.int32)
        shuf = tuple(lane ^ sh for sh in (1, 2, 4, 8))
        tile_base = wid * rowtiles_per_w
        nch_total = rowtiles_per_w * nchunk

        def chunk_src(g):
            gc = jnp.minimum(g, nch_total - 1)
            rt = tile_base + gc // nchunk
            return probs_hbm.at[pl.ds(rt * _TR, _TR),
                                pl.ds((gc % nchunk) * _CCOLS, _CCOLS)]

        pltpu.async_copy(chunk_src(0), buf0, sem0)

        def scan_chunk(g, buf, sem, nbuf, nsem, accs):
            pltpu.async_copy(chunk_src(g + 1), nbuf, nsem)
            pltpu.make_async_copy(chunk_src(g), buf, sem).wait()
            k = g % nchunk

            def vec_body(i, c2):
                vs = list(c2[:_TR])
                vt = list(c2[_TR:])
                tcur = k * inner_iters + i
                for r in range(_TR):
                    x = buf[r, pl.ds(i * _L, _L)]
                    m = x > vs[r]
                    vs[r] = jnp.where(m, x, vs[r])
                    vt[r] = jnp.where(m, jnp.broadcast_to(tcur, (_L,)), vt[r])
                return tuple(vs) + tuple(vt)

            return lax.fori_loop(0, inner_iters, vec_body, accs)

        def rowtile_body(rt, carry):
            res, resm = carry
            accs = tuple([neg] * _TR) + tuple([zero] * _TR)

            def pair_body(kk, accs):
                g = rt * nchunk + 2 * kk
                accs = scan_chunk(g, buf0, sem0, buf1, sem1, accs)
                accs = scan_chunk(g + 1, buf1, sem1, buf0, sem0, accs)
                return accs

            accs = lax.fori_loop(0, nchunk // 2, pair_body, accs)
            vs, vt = accs[:_TR], accs[_TR:]
            for r in range(_TR):
                rmax = vs[r]
                for sv in shuf:
                    rmax = jnp.maximum(
                        rmax, rmax.at[sv].get(mode='promise_in_bounds'))
                idx = vt[r] * _L + lane
                cand = jnp.where(vs[r] == rmax, idx, big)
                for sv in shuf:
                    cand = jnp.minimum(
                        cand, cand.at[sv].get(mode='promise_in_bounds'))
                rl = rt * _TR + r
                res = jnp.where(lane == rl, cand, res)
                resm = jnp.where(lane == rl, rmax, resm)
            return (res, resm)

        res, resm = lax.fori_loop(0, rowtiles_per_w, rowtile_body,
                                  (zero, neg))
        pltpu.make_async_copy(chunk_src(nch_total), buf0, sem0).wait()
        outv[...] = res
        outm[...] = resm
        pltpu.sync_copy(outv, out_idx.at[pl.ds(wid * rows_per_w, rows_per_w)])
        pltpu.sync_copy(outm, out_max.at[pl.ds(wid * rows_per_w, rows_per_w)])

    return body(probs)


def _tail_merge_tc(tail, sc_arg2d, sc_max2d, col0):
    num_rows, tcols = tail.shape

    def body(tail_ref, sca_ref, scm_ref, tam_ref):
        t = tail_ref[...]
        iot = lax.broadcasted_iota(jnp.int32, (num_rows, tcols), 1)
        tmax = jnp.max(t, axis=1, keepdims=True)
        targ = jnp.min(jnp.where(t == tmax, iot, 2**30),
                       axis=1, keepdims=True) + col0
        use_tail = tmax > scm_ref[...]
        tam_ref[...] = jnp.where(use_tail, targ, sca_ref[...])

    return pl.pallas_call(
        body,
        out_shape=jax.ShapeDtypeStruct((num_rows, 1), jnp.int32),
    )(tail, sc_arg2d, sc_max2d)


def _epilogue_tc(tam2d, draft2d, cu2d, bonus, spec):
    batch = tam2d.shape[0]
    sentinel = 2 * spec

    def body(tam_ref, draft_ref, cu_ref, bonus_ref, out_ref):
        tam = tam_ref[...]
        draft = draft_ref[...]
        cu = cu_ref[...]
        bns = bonus_ref[...]
        cu_prev = jnp.concatenate(
            [jnp.zeros((1, 1), jnp.int32), cu[:-1, :]], axis=0)
        dpr = cu - cu_prev
        pos = lax.broadcasted_iota(jnp.int32, (batch, spec), 1)
        mm_pos = jnp.where(draft != tam, pos, sentinel)
        fm = jnp.min(mm_pos, axis=1, keepdims=True)
        fm = jnp.where(fm == sentinel, dpr, fm)
        copy_len = jnp.minimum(fm + 1, dpr)
        main = jnp.where(pos < copy_len, tam, -1)
        bcol = jnp.where(fm >= dpr, bns, -1)
        out_ref[...] = jnp.concatenate([main, bcol], axis=1)

    return pl.pallas_call(
        body,
        out_shape=jax.ShapeDtypeStruct((batch, spec + 1), jnp.int32),
    )(tam2d, draft2d, cu2d, bonus)


def kernel(draft_token_ids, cu_num_draft_tokens, target_probs,
           bonus_token_ids, max_spec_len):
    num_tokens, vocab = target_probs.shape
    batch = cu_num_draft_tokens.shape[0]
    spec = num_tokens // batch
    sc_cols = (vocab // _CCOLS) * _CCOLS
    sc_arg, sc_max = _argmax_sc(target_probs, num_tokens, sc_cols)
    tam2d = _tail_merge_tc(
        target_probs[:, sc_cols:],
        sc_arg.reshape(num_tokens, 1),
        sc_max.reshape(num_tokens, 1),
        sc_cols)
    return _epilogue_tc(
        tam2d.reshape(batch, spec),
        draft_token_ids.reshape(batch, spec),
        cu_num_draft_tokens.reshape(batch, 1).astype(jnp.int32),
        bonus_token_ids.astype(jnp.int32),
        spec)


# speedup vs baseline: 6.0072x; 2.6934x over previous
import functools

import jax
import jax.numpy as jnp
from jax import lax
from jax.experimental import pallas as pl
from jax.experimental.pallas import tpu as pltpu
from jax.experimental.pallas import tpu_sc as plsc

_NC = 2
_NS = 16
_L = 16

_BAND = 128
_NRANGE = 8
_CCOLS = 368


def _argmax_sc(probs_t, vocab, num_rows):
    nbands = num_rows // _BAND
    range_cols = -(-vocab // _NRANGE)
    range_cols = -(-range_cols // 8) * 8
    nchunk = -(-range_cols // _CCOLS)
    nchunk += nchunk % 2
    last_start = vocab - _CCOLS
    npart = _NRANGE * num_rows
    mesh = plsc.VectorSubcoreMesh(
        core_axis_name="c", subcore_axis_name="s",
        num_cores=_NC, num_subcores=_NS)

    @functools.partial(
        pl.kernel,
        out_type=(jax.ShapeDtypeStruct((npart,), jnp.float32),
                  jax.ShapeDtypeStruct((npart,), jnp.int32)),
        mesh=mesh,
        scratch_types=[
            pltpu.VMEM((_CCOLS, _BAND), jnp.float32),
            pltpu.VMEM((_CCOLS, _BAND), jnp.float32),
            pltpu.VMEM((_BAND,), jnp.float32),
            pltpu.VMEM((_BAND,), jnp.int32),
            pltpu.SemaphoreType.DMA,
            pltpu.SemaphoreType.DMA,
        ],
        compiler_params=pltpu.CompilerParams(use_tc_tiling_on_sc=True),
    )
    def body(pt_hbm, out_val, out_idx, buf0, buf1, stgv, stgi, sem0, sem1):
        c = lax.axis_index("c")
        s = lax.axis_index("s")
        band = nbands // _NC * c + s // _NRANGE
        rng = s % _NRANGE
        col0 = rng * range_cols
        lane = lax.broadcasted_iota(jnp.int32, (_L,), 0)
        neg = jnp.full((_L,), -1.0, jnp.float32)
        zero = jnp.zeros((_L,), jnp.int32)
        ngroups = _BAND // _L

        def chunk_src(k):
            start = jnp.minimum(col0 + k * _CCOLS, last_start)
            return pt_hbm.at[pl.ds(start, _CCOLS), pl.ds(band * _BAND, _BAND)]

        pltpu.async_copy(chunk_src(0), buf0, sem0)

        def scan_chunk(k, buf, sem, nbuf, nsem, accs):
            pltpu.async_copy(chunk_src(k + 1), nbuf, nsem)
            pltpu.make_async_copy(chunk_src(k), buf, sem).wait()
            cbase = jnp.minimum(col0 + k * _CCOLS, last_start)

            def col_body(cc, a2):
                vs = list(a2[:ngroups])
                vi = list(a2[ngroups:])
                colidx = cbase + cc
                for r in range(ngroups):
                    x = buf[cc, pl.ds(r * _L, _L)]
                    m = x > vs[r]
                    vs[r] = jnp.where(m, x, vs[r])
                    vi[r] = jnp.where(
                        m, jnp.broadcast_to(colidx, (_L,)), vi[r])
                return tuple(vs) + tuple(vi)

            return lax.fori_loop(0, _CCOLS, col_body, accs)

        def pair_body(kk, accs):
            accs = scan_chunk(2 * kk, buf0, sem0, buf1, sem1, accs)
            accs = scan_chunk(2 * kk + 1, buf1, sem1, buf0, sem0, accs)
            return accs

        init = tuple([neg] * ngroups) + tuple([zero] * ngroups)
        accs = lax.fori_loop(0, nchunk // 2, pair_body, init)
        pltpu.make_async_copy(chunk_src(nchunk), buf0, sem0).wait()
        for r in range(ngroups):
            stgv[pl.ds(r * _L, _L)] = accs[r]
            stgi[pl.ds(r * _L, _L)] = accs[ngroups + r]
        off = rng * num_rows + band * _BAND
        pltpu.sync_copy(stgv, out_val.at[pl.ds(off, _BAND)])
        pltpu.sync_copy(stgi, out_idx.at[pl.ds(off, _BAND)])

    return body(probs_t)


def _merge_tc(val2, idx2):
    nrange, num_rows = val2.shape

    def body(val_ref, idx_ref, tam_ref):
        v = val_ref[...]
        i = idx_ref[...]
        best = jnp.max(v, axis=0, keepdims=True)
        tam_ref[...] = jnp.min(
            jnp.where(v == best, i, 2**30), axis=0, keepdims=True)

    return pl.pallas_call(
        body,
        out_shape=jax.ShapeDtypeStruct((1, num_rows), jnp.int32),
    )(val2, idx2)


def _epilogue_tc(tam2d, draft2d, cu2d, bonus, spec):
    batch = tam2d.shape[0]
    sentinel = 2 * spec

    def body(tam_ref, draft_ref, cu_ref, bonus_ref, out_ref):
        tam = tam_ref[...]
        draft = draft_ref[...]
        cu = cu_ref[...]
        bns = bonus_ref[...]
        cu_prev = jnp.concatenate(
            [jnp.zeros((1, 1), jnp.int32), cu[:-1, :]], axis=0)
        dpr = cu - cu_prev
        pos = lax.broadcasted_iota(jnp.int32, (batch, spec), 1)
        mm_pos = jnp.where(draft != tam, pos, sentinel)
        fm = jnp.min(mm_pos, axis=1, keepdims=True)
        fm = jnp.where(fm == sentinel, dpr, fm)
        copy_len = jnp.minimum(fm + 1, dpr)
        main = jnp.where(pos < copy_len, tam, -1)
        bcol = jnp.where(fm >= dpr, bns, -1)
        out_ref[...] = jnp.concatenate([main, bcol], axis=1)

    return pl.pallas_call(
        body,
        out_shape=jax.ShapeDtypeStruct((batch, spec + 1), jnp.int32),
    )(tam2d, draft2d, cu2d, bonus)


def kernel(draft_token_ids, cu_num_draft_tokens, target_probs,
           bonus_token_ids, max_spec_len):
    num_tokens, vocab = target_probs.shape
    batch = cu_num_draft_tokens.shape[0]
    spec = num_tokens // batch
    val_p, idx_p = _argmax_sc(target_probs.T, vocab, num_tokens)
    tam = _merge_tc(val_p.reshape(_NRANGE, num_tokens),
                    idx_p.reshape(_NRANGE, num_tokens))
    return _epilogue_tc(
        tam.reshape(batch, spec),
        draft_token_ids.reshape(batch, spec),
        cu_num_draft_tokens.reshape(batch, 1).astype(jnp.int32),
        bonus_token_ids.astype(jnp.int32),
        spec)


# speedup vs baseline: 6.7105x; 1.1171x over previous
import functools

import jax
import jax.numpy as jnp
from jax import lax
from jax.experimental import pallas as pl
from jax.experimental.pallas import tpu as pltpu
from jax.experimental.pallas import tpu_sc as plsc

_NC = 2
_NS = 16
_L = 16

_BAND = 128
_NRANGE = 8
_CCOLS = 368

_TC_BLK = 2048
_SC_COLS = 25 * _TC_BLK


def _argmax_sc(probs_t, vocab, num_rows):
    nbands = num_rows // _BAND
    range_cols = -(-vocab // _NRANGE)
    range_cols = -(-range_cols // 8) * 8
    nchunk = -(-range_cols // _CCOLS)
    nchunk += nchunk % 2
    last_start = vocab - _CCOLS
    npart = _NRANGE * num_rows
    mesh = plsc.VectorSubcoreMesh(
        core_axis_name="c", subcore_axis_name="s",
        num_cores=_NC, num_subcores=_NS)

    @functools.partial(
        pl.kernel,
        out_type=(jax.ShapeDtypeStruct((npart,), jnp.float32),
                  jax.ShapeDtypeStruct((npart,), jnp.int32)),
        mesh=mesh,
        scratch_types=[
            pltpu.VMEM((_CCOLS, _BAND), jnp.float32),
            pltpu.VMEM((_CCOLS, _BAND), jnp.float32),
            pltpu.VMEM((_BAND,), jnp.float32),
            pltpu.VMEM((_BAND,), jnp.int32),
            pltpu.SemaphoreType.DMA,
            pltpu.SemaphoreType.DMA,
        ],
        compiler_params=pltpu.CompilerParams(use_tc_tiling_on_sc=True),
    )
    def body(pt_hbm, out_val, out_idx, buf0, buf1, stgv, stgi, sem0, sem1):
        c = lax.axis_index("c")
        s = lax.axis_index("s")
        band = nbands // _NC * c + s // _NRANGE
        rng = s % _NRANGE
        col0 = rng * range_cols
        lane = lax.broadcasted_iota(jnp.int32, (_L,), 0)
        neg = jnp.full((_L,), -1.0, jnp.float32)
        zero = jnp.zeros((_L,), jnp.int32)
        ngroups = _BAND // _L

        def chunk_src(k):
            start = jnp.minimum(col0 + k * _CCOLS, last_start)
            return pt_hbm.at[pl.ds(start, _CCOLS), pl.ds(band * _BAND, _BAND)]

        pltpu.async_copy(chunk_src(0), buf0, sem0)

        def scan_chunk(k, buf, sem, nbuf, nsem, accs):
            pltpu.async_copy(chunk_src(k + 1), nbuf, nsem)
            pltpu.make_async_copy(chunk_src(k), buf, sem).wait()
            cbase = jnp.minimum(col0 + k * _CCOLS, last_start)

            def col_body(cc, a2):
                vs = list(a2[:ngroups])
                vi = list(a2[ngroups:])
                colidx = cbase + cc
                for r in range(ngroups):
                    x = buf[cc, pl.ds(r * _L, _L)]
                    m = x > vs[r]
                    vs[r] = jnp.where(m, x, vs[r])
                    vi[r] = jnp.where(
                        m, jnp.broadcast_to(colidx, (_L,)), vi[r])
                return tuple(vs) + tuple(vi)

            return lax.fori_loop(0, _CCOLS, col_body, accs)

        def pair_body(kk, accs):
            accs = scan_chunk(2 * kk, buf0, sem0, buf1, sem1, accs)
            accs = scan_chunk(2 * kk + 1, buf1, sem1, buf0, sem0, accs)
            return accs

        init = tuple([neg] * ngroups) + tuple([zero] * ngroups)
        accs = lax.fori_loop(0, nchunk // 2, pair_body, init)
        pltpu.make_async_copy(chunk_src(nchunk), buf0, sem0).wait()
        for r in range(ngroups):
            stgv[pl.ds(r * _L, _L)] = accs[r]
            stgi[pl.ds(r * _L, _L)] = accs[ngroups + r]
        off = rng * num_rows + band * _BAND
        pltpu.sync_copy(stgv, out_val.at[pl.ds(off, _BAND)])
        pltpu.sync_copy(stgi, out_idx.at[pl.ds(off, _BAND)])

    return body(probs_t)


def _argmax_tc(probs_t, col_start, vocab, num_rows):
    grid = -(-(vocab - col_start) // _TC_BLK)

    def body(x_ref, val_ref, idx_ref, vacc, iacc):
        k = pl.program_id(0)
        cols = lax.broadcasted_iota(jnp.int32, (_TC_BLK, num_rows), 0) + (
            col_start + k * _TC_BLK)
        x = jnp.where(cols < vocab, x_ref[...], -1.0)
        bm = jnp.max(x, axis=0, keepdims=True)
        bi = jnp.min(jnp.where(x == bm, cols, 2**30), axis=0, keepdims=True)

        @pl.when(k == 0)
        def _():
            vacc[...] = bm
            iacc[...] = bi

        @pl.when(k > 0)
        def _():
            better = bm > vacc[...]
            vacc[...] = jnp.where(better, bm, vacc[...])
            iacc[...] = jnp.where(better, bi, iacc[...])

        @pl.when(k == grid - 1)
        def _():
            val_ref[...] = vacc[...]
            idx_ref[...] = iacc[...]

    return pl.pallas_call(
        body,
        grid=(grid,),
        in_specs=[pl.BlockSpec((_TC_BLK, num_rows),
                               lambda k: (col_start // _TC_BLK + k, 0))],
        out_specs=(pl.BlockSpec((1, num_rows), lambda k: (0, 0)),
                   pl.BlockSpec((1, num_rows), lambda k: (0, 0))),
        out_shape=(jax.ShapeDtypeStruct((1, num_rows), jnp.float32),
                   jax.ShapeDtypeStruct((1, num_rows), jnp.int32)),
        scratch_shapes=[pltpu.VMEM((1, num_rows), jnp.float32),
                        pltpu.VMEM((1, num_rows), jnp.int32)],
    )(probs_t)


def _merge_tc(val2, idx2, val1, idx1):
    nrange, num_rows = val2.shape

    def body(val_ref, idx_ref, v1_ref, i1_ref, tam_ref):
        v = val_ref[...]
        i = idx_ref[...]
        v1 = v1_ref[...]
        i1 = i1_ref[...]
        best = jnp.maximum(jnp.max(v, axis=0, keepdims=True), v1)
        cand = jnp.min(
            jnp.where(v == best, i, 2**30), axis=0, keepdims=True)
        tam_ref[...] = jnp.minimum(
            cand, jnp.where(v1 == best, i1, 2**30))

    return pl.pallas_call(
        body,
        out_shape=jax.ShapeDtypeStruct((1, num_rows), jnp.int32),
    )(val2, idx2, val1, idx1)


def _epilogue_tc(tam2d, draft2d, cu2d, bonus, spec):
    batch = tam2d.shape[0]
    sentinel = 2 * spec

    def body(tam_ref, draft_ref, cu_ref, bonus_ref, out_ref):
        tam = tam_ref[...]
        draft = draft_ref[...]
        cu = cu_ref[...]
        bns = bonus_ref[...]
        cu_prev = jnp.concatenate(
            [jnp.zeros((1, 1), jnp.int32), cu[:-1, :]], axis=0)
        dpr = cu - cu_prev
        pos = lax.broadcasted_iota(jnp.int32, (batch, spec), 1)
        mm_pos = jnp.where(draft != tam, pos, sentinel)
        fm = jnp.min(mm_pos, axis=1, keepdims=True)
        fm = jnp.where(fm == sentinel, dpr, fm)
        copy_len = jnp.minimum(fm + 1, dpr)
        main = jnp.where(pos < copy_len, tam, -1)
        bcol = jnp.where(fm >= dpr, bns, -1)
        out_ref[...] = jnp.concatenate([main, bcol], axis=1)

    return pl.pallas_call(
        body,
        out_shape=jax.ShapeDtypeStruct((batch, spec + 1), jnp.int32),
    )(tam2d, draft2d, cu2d, bonus)


def kernel(draft_token_ids, cu_num_draft_tokens, target_probs,
           bonus_token_ids, max_spec_len):
    num_tokens, vocab = target_probs.shape
    batch = cu_num_draft_tokens.shape[0]
    spec = num_tokens // batch
    probs_t = target_probs.T
    val_p, idx_p = _argmax_sc(probs_t, _SC_COLS, num_tokens)
    val_t, idx_t = _argmax_tc(probs_t, _SC_COLS, vocab, num_tokens)
    tam = _merge_tc(val_p.reshape(_NRANGE, num_tokens),
                    idx_p.reshape(_NRANGE, num_tokens),
                    val_t, idx_t)
    return _epilogue_tc(
        tam.reshape(batch, spec),
        draft_token_ids.reshape(batch, spec),
        cu_num_draft_tokens.reshape(batch, 1).astype(jnp.int32),
        bonus_token_ids.astype(jnp.int32),
        spec)


# speedup vs baseline: 7.0390x; 1.0489x over previous
import functools

import jax
import jax.numpy as jnp
from jax import lax
from jax.experimental import pallas as pl
from jax.experimental.pallas import tpu as pltpu
from jax.experimental.pallas import tpu_sc as plsc

_NC = 2
_NS = 16
_L = 16

_BAND = 128
_NRANGE = 8
_CCOLS = 368

_TC_BLK = 4096
_SC_COLS = 12 * _TC_BLK


def _argmax_sc(probs_t, vocab, num_rows):
    nbands = num_rows // _BAND
    range_cols = -(-vocab // _NRANGE)
    range_cols = -(-range_cols // 8) * 8
    nchunk = -(-range_cols // _CCOLS)
    nchunk += nchunk % 2
    last_start = vocab - _CCOLS
    npart = _NRANGE * num_rows
    mesh = plsc.VectorSubcoreMesh(
        core_axis_name="c", subcore_axis_name="s",
        num_cores=_NC, num_subcores=_NS)

    @functools.partial(
        pl.kernel,
        out_type=(jax.ShapeDtypeStruct((npart,), jnp.float32),
                  jax.ShapeDtypeStruct((npart,), jnp.int32)),
        mesh=mesh,
        scratch_types=[
            pltpu.VMEM((_CCOLS, _BAND), jnp.float32),
            pltpu.VMEM((_CCOLS, _BAND), jnp.float32),
            pltpu.VMEM((_BAND,), jnp.float32),
            pltpu.VMEM((_BAND,), jnp.int32),
            pltpu.SemaphoreType.DMA,
            pltpu.SemaphoreType.DMA,
        ],
        compiler_params=pltpu.CompilerParams(use_tc_tiling_on_sc=True),
    )
    def body(pt_hbm, out_val, out_idx, buf0, buf1, stgv, stgi, sem0, sem1):
        c = lax.axis_index("c")
        s = lax.axis_index("s")
        band = nbands // _NC * c + s // _NRANGE
        rng = s % _NRANGE
        col0 = rng * range_cols
        lane = lax.broadcasted_iota(jnp.int32, (_L,), 0)
        neg = jnp.full((_L,), -1.0, jnp.float32)
        zero = jnp.zeros((_L,), jnp.int32)
        ngroups = _BAND // _L

        def chunk_src(k):
            start = jnp.minimum(col0 + k * _CCOLS, last_start)
            return pt_hbm.at[pl.ds(start, _CCOLS), pl.ds(band * _BAND, _BAND)]

        pltpu.async_copy(chunk_src(0), buf0, sem0)

        def scan_chunk(k, buf, sem, nbuf, nsem, accs):
            pltpu.async_copy(chunk_src(k + 1), nbuf, nsem)
            pltpu.make_async_copy(chunk_src(k), buf, sem).wait()
            cbase = jnp.minimum(col0 + k * _CCOLS, last_start)

            def col_body(cc, a2):
                vs = list(a2[:ngroups])
                vi = list(a2[ngroups:])
                colidx = cbase + cc
                for r in range(ngroups):
                    x = buf[cc, pl.ds(r * _L, _L)]
                    m = x > vs[r]
                    vs[r] = jnp.where(m, x, vs[r])
                    vi[r] = jnp.where(
                        m, jnp.broadcast_to(colidx, (_L,)), vi[r])
                return tuple(vs) + tuple(vi)

            return lax.fori_loop(0, _CCOLS, col_body, accs)

        def pair_body(kk, accs):
            accs = scan_chunk(2 * kk, buf0, sem0, buf1, sem1, accs)
            accs = scan_chunk(2 * kk + 1, buf1, sem1, buf0, sem0, accs)
            return accs

        init = tuple([neg] * ngroups) + tuple([zero] * ngroups)
        accs = lax.fori_loop(0, nchunk // 2, pair_body, init)
        pltpu.make_async_copy(chunk_src(nchunk), buf0, sem0).wait()
        for r in range(ngroups):
            stgv[pl.ds(r * _L, _L)] = accs[r]
            stgi[pl.ds(r * _L, _L)] = accs[ngroups + r]
        off = rng * num_rows + band * _BAND
        pltpu.sync_copy(stgv, out_val.at[pl.ds(off, _BAND)])
        pltpu.sync_copy(stgi, out_idx.at[pl.ds(off, _BAND)])

    return body(probs_t)


def _argmax_tc(probs_t, col_start, vocab, num_rows):
    grid = -(-(vocab - col_start) // _TC_BLK)

    def body(x_ref, val_ref, idx_ref, vacc, iacc):
        k = pl.program_id(0)
        cols = lax.broadcasted_iota(jnp.int32, (_TC_BLK, num_rows), 0) + (
            col_start + k * _TC_BLK)
        x = jnp.where(cols < vocab, x_ref[...], -1.0)
        bm = jnp.max(x, axis=0, keepdims=True)
        bi = jnp.argmax(x, axis=0, keepdims=True).astype(jnp.int32) + (
            col_start + k * _TC_BLK)

        @pl.when(k == 0)
        def _():
            vacc[...] = bm
            iacc[...] = bi

        @pl.when(k > 0)
        def _():
            better = bm > vacc[...]
            vacc[...] = jnp.where(better, bm, vacc[...])
            iacc[...] = jnp.where(better, bi, iacc[...])

        @pl.when(k == grid - 1)
        def _():
            val_ref[...] = vacc[...]
            idx_ref[...] = iacc[...]

    return pl.pallas_call(
        body,
        grid=(grid,),
        in_specs=[pl.BlockSpec((_TC_BLK, num_rows),
                               lambda k: (col_start // _TC_BLK + k, 0))],
        out_specs=(pl.BlockSpec((1, num_rows), lambda k: (0, 0)),
                   pl.BlockSpec((1, num_rows), lambda k: (0, 0))),
        out_shape=(jax.ShapeDtypeStruct((1, num_rows), jnp.float32),
                   jax.ShapeDtypeStruct((1, num_rows), jnp.int32)),
        scratch_shapes=[pltpu.VMEM((1, num_rows), jnp.float32),
                        pltpu.VMEM((1, num_rows), jnp.int32)],
    )(probs_t)


def _merge_tc(val_p, idx_p, val1, idx1, num_rows):
    def body(v8_ref, i8_ref, v1_ref, i1_ref, tam_ref):
        vs = [v8_ref[pl.ds(r * num_rows, num_rows)] for r in range(_NRANGE)]
        is_ = [i8_ref[pl.ds(r * num_rows, num_rows)] for r in range(_NRANGE)]
        vs.append(v1_ref[...])
        is_.append(i1_ref[...])
        best = vs[0]
        for v in vs[1:]:
            best = jnp.maximum(best, v)
        cand = jnp.where(vs[0] == best, is_[0], 2**30)
        for v, i in zip(vs[1:], is_[1:]):
            cand = jnp.minimum(cand, jnp.where(v == best, i, 2**30))
        tam_ref[...] = cand

    return pl.pallas_call(
        body,
        out_shape=jax.ShapeDtypeStruct((num_rows,), jnp.int32),
    )(val_p, idx_p, val1, idx1)


def _epilogue_tc(tam2d, draft2d, cu2d, bonus, spec):
    batch = tam2d.shape[0]
    sentinel = 2 * spec

    def body(tam_ref, draft_ref, cu_ref, bonus_ref, out_ref):
        tam = tam_ref[...]
        draft = draft_ref[...]
        cu = cu_ref[...]
        bns = bonus_ref[...]
        cu_prev = jnp.concatenate(
            [jnp.zeros((1, 1), jnp.int32), cu[:-1, :]], axis=0)
        dpr = cu - cu_prev
        pos = lax.broadcasted_iota(jnp.int32, (batch, spec), 1)
        mm_pos = jnp.where(draft != tam, pos, sentinel)
        fm = jnp.min(mm_pos, axis=1, keepdims=True)
        fm = jnp.where(fm == sentinel, dpr, fm)
        copy_len = jnp.minimum(fm + 1, dpr)
        main = jnp.where(pos < copy_len, tam, -1)
        bcol = jnp.where(fm >= dpr, bns, -1)
        out_ref[...] = jnp.concatenate([main, bcol], axis=1)

    return pl.pallas_call(
        body,
        out_shape=jax.ShapeDtypeStruct((batch, spec + 1), jnp.int32),
    )(tam2d, draft2d, cu2d, bonus)


def kernel(draft_token_ids, cu_num_draft_tokens, target_probs,
           bonus_token_ids, max_spec_len):
    num_tokens, vocab = target_probs.shape
    batch = cu_num_draft_tokens.shape[0]
    spec = num_tokens // batch
    probs_t = target_probs.T
    val_p, idx_p = _argmax_sc(probs_t, _SC_COLS, num_tokens)
    val_t, idx_t = _argmax_tc(probs_t, _SC_COLS, vocab, num_tokens)
    tam = _merge_tc(val_p, idx_p,
                    val_t.reshape(num_tokens), idx_t.reshape(num_tokens),
                    num_tokens)
    return _epilogue_tc(
        tam.reshape(batch, spec),
        draft_token_ids.reshape(batch, spec),
        cu_num_draft_tokens.reshape(batch, 1).astype(jnp.int32),
        bonus_token_ids.astype(jnp.int32),
        spec)


# speedup vs baseline: 7.1008x; 1.0088x over previous
import functools

import jax
import jax.numpy as jnp
from jax import lax
from jax.experimental import pallas as pl
from jax.experimental.pallas import tpu as pltpu
from jax.experimental.pallas import tpu_sc as plsc

_NC = 2
_NS = 16
_L = 16

_BAND = 128
_NRANGE = 8
_CCOLS = 368

_TC_BLK = 4096
_SC_COLS = 11 * _TC_BLK


def _argmax_sc(probs_t, vocab, num_rows):
    nbands = num_rows // _BAND
    range_cols = -(-vocab // _NRANGE)
    range_cols = -(-range_cols // 8) * 8
    nchunk = -(-range_cols // _CCOLS)
    nchunk += nchunk % 2
    last_start = vocab - _CCOLS
    npart = _NRANGE * num_rows
    mesh = plsc.VectorSubcoreMesh(
        core_axis_name="c", subcore_axis_name="s",
        num_cores=_NC, num_subcores=_NS)

    @functools.partial(
        pl.kernel,
        out_type=(jax.ShapeDtypeStruct((npart,), jnp.float32),
                  jax.ShapeDtypeStruct((npart,), jnp.int32)),
        mesh=mesh,
        scratch_types=[
            pltpu.VMEM((_CCOLS, _BAND), jnp.float32),
            pltpu.VMEM((_CCOLS, _BAND), jnp.float32),
            pltpu.VMEM((_BAND,), jnp.float32),
            pltpu.VMEM((_BAND,), jnp.int32),
            pltpu.SemaphoreType.DMA,
            pltpu.SemaphoreType.DMA,
        ],
        compiler_params=pltpu.CompilerParams(use_tc_tiling_on_sc=True),
    )
    def body(pt_hbm, out_val, out_idx, buf0, buf1, stgv, stgi, sem0, sem1):
        c = lax.axis_index("c")
        s = lax.axis_index("s")
        band = nbands // _NC * c + s // _NRANGE
        rng = s % _NRANGE
        col0 = rng * range_cols
        lane = lax.broadcasted_iota(jnp.int32, (_L,), 0)
        neg = jnp.full((_L,), -1.0, jnp.float32)
        zero = jnp.zeros((_L,), jnp.int32)
        ngroups = _BAND // _L

        def chunk_src(k):
            start = jnp.minimum(col0 + k * _CCOLS, last_start)
            return pt_hbm.at[pl.ds(start, _CCOLS), pl.ds(band * _BAND, _BAND)]

        pltpu.async_copy(chunk_src(0), buf0, sem0)

        def scan_chunk(k, buf, sem, nbuf, nsem, accs):
            pltpu.async_copy(chunk_src(k + 1), nbuf, nsem)
            pltpu.make_async_copy(chunk_src(k), buf, sem).wait()
            cbase = jnp.minimum(col0 + k * _CCOLS, last_start)

            def col_body(cc, a2):
                vs = list(a2[:ngroups])
                vi = list(a2[ngroups:])
                colidx = cbase + cc
                for r in range(ngroups):
                    x = buf[cc, pl.ds(r * _L, _L)]
                    m = x > vs[r]
                    vs[r] = jnp.where(m, x, vs[r])
                    vi[r] = jnp.where(
                        m, jnp.broadcast_to(colidx, (_L,)), vi[r])
                return tuple(vs) + tuple(vi)

            return lax.fori_loop(0, _CCOLS, col_body, accs)

        def pair_body(kk, accs):
            accs = scan_chunk(2 * kk, buf0, sem0, buf1, sem1, accs)
            accs = scan_chunk(2 * kk + 1, buf1, sem1, buf0, sem0, accs)
            return accs

        init = tuple([neg] * ngroups) + tuple([zero] * ngroups)
        accs = lax.fori_loop(0, nchunk // 2, pair_body, init)
        pltpu.make_async_copy(chunk_src(nchunk), buf0, sem0).wait()
        for r in range(ngroups):
            stgv[pl.ds(r * _L, _L)] = accs[r]
            stgi[pl.ds(r * _L, _L)] = accs[ngroups + r]
        off = rng * num_rows + band * _BAND
        pltpu.sync_copy(stgv, out_val.at[pl.ds(off, _BAND)])
        pltpu.sync_copy(stgi, out_idx.at[pl.ds(off, _BAND)])

    return body(probs_t)


def _argmax_tc(probs_t, col_start, vocab, num_rows):
    grid = -(-(vocab - col_start) // _TC_BLK)

    def body(x_ref, val_ref, idx_ref, vacc, iacc):
        k = pl.program_id(0)
        cols = lax.broadcasted_iota(jnp.int32, (_TC_BLK, num_rows), 0) + (
            col_start + k * _TC_BLK)
        x = jnp.where(cols < vocab, x_ref[...], -1.0)
        bm = jnp.max(x, axis=0, keepdims=True)
        bi = jnp.argmax(x, axis=0, keepdims=True).astype(jnp.int32) + (
            col_start + k * _TC_BLK)

        @pl.when(k == 0)
        def _():
            vacc[...] = bm
            iacc[...] = bi

        @pl.when(k > 0)
        def _():
            better = bm > vacc[...]
            vacc[...] = jnp.where(better, bm, vacc[...])
            iacc[...] = jnp.where(better, bi, iacc[...])

        @pl.when(k == grid - 1)
        def _():
            val_ref[...] = vacc[...]
            idx_ref[...] = iacc[...]

    return pl.pallas_call(
        body,
        grid=(grid,),
        in_specs=[pl.BlockSpec((_TC_BLK, num_rows),
                               lambda k: (col_start // _TC_BLK + k, 0))],
        out_specs=(pl.BlockSpec((1, num_rows), lambda k: (0, 0)),
                   pl.BlockSpec((1, num_rows), lambda k: (0, 0))),
        out_shape=(jax.ShapeDtypeStruct((1, num_rows), jnp.float32),
                   jax.ShapeDtypeStruct((1, num_rows), jnp.int32)),
        scratch_shapes=[pltpu.VMEM((1, num_rows), jnp.float32),
                        pltpu.VMEM((1, num_rows), jnp.int32)],
    )(probs_t)


def _merge_epilogue_tc(val_p, idx_p, val1, idx1, draft, cu2d, bonus,
                       num_rows, spec):
    batch = num_rows // spec
    sentinel = 2 * spec

    def body(v8_ref, i8_ref, v1_ref, i1_ref, tam_ref):
        vs = [v8_ref[pl.ds(r * num_rows, num_rows)] for r in range(_NRANGE)]
        is_ = [i8_ref[pl.ds(r * num_rows, num_rows)] for r in range(_NRANGE)]
        vs.append(v1_ref[...])
        is_.append(i1_ref[...])
        best = vs[0]
        for v in vs[1:]:
            best = jnp.maximum(best, v)
        cand = jnp.where(vs[0] == best, is_[0], 2**30)
        for v, i in zip(vs[1:], is_[1:]):
            cand = jnp.minimum(cand, jnp.where(v == best, i, 2**30))
        tam_ref[...] = cand

    tam = pl.pallas_call(
        body,
        out_shape=jax.ShapeDtypeStruct((num_rows,), jnp.int32),
    )(val_p, idx_p, val1, idx1)

    def body2(tam_ref, draft_ref, cu_ref, bonus_ref, out_ref):
        tam = tam_ref[...]
        draft = draft_ref[...]
        cu = cu_ref[...]
        bns = bonus_ref[...]
        cu_prev = jnp.concatenate(
            [jnp.zeros((1, 1), jnp.int32), cu[:-1, :]], axis=0)
        dpr = cu - cu_prev
        pos = lax.broadcasted_iota(jnp.int32, (batch, spec), 1)
        mm_pos = jnp.where(draft != tam, pos, sentinel)
        fm = jnp.min(mm_pos, axis=1, keepdims=True)
        fm = jnp.where(fm == sentinel, dpr, fm)
        copy_len = jnp.minimum(fm + 1, dpr)
        main = jnp.where(pos < copy_len, tam, -1)
        bcol = jnp.where(fm >= dpr, bns, -1)
        out_ref[...] = jnp.concatenate([main, bcol], axis=1)

    return pl.pallas_call(
        body2,
        out_shape=jax.ShapeDtypeStruct((batch, spec + 1), jnp.int32),
    )(tam.reshape(batch, spec), draft.reshape(batch, spec), cu2d, bonus)


def kernel(draft_token_ids, cu_num_draft_tokens, target_probs,
           bonus_token_ids, max_spec_len):
    num_tokens, vocab = target_probs.shape
    batch = cu_num_draft_tokens.shape[0]
    spec = num_tokens // batch
    probs_t = target_probs.T
    val_p, idx_p = _argmax_sc(probs_t, _SC_COLS, num_tokens)
    val_t, idx_t = _argmax_tc(probs_t, _SC_COLS, vocab, num_tokens)
    return _merge_epilogue_tc(
        val_p, idx_p,
        val_t.reshape(num_tokens), idx_t.reshape(num_tokens),
        draft_token_ids,
        cu_num_draft_tokens.reshape(batch, 1).astype(jnp.int32),
        bonus_token_ids.astype(jnp.int32),
        num_tokens, spec)
